# initial kernel scaffold (unmeasured)
import jax
import jax.numpy as jnp
from jax import lax
from jax.experimental import pallas as pl
from jax.experimental.pallas import tpu as pltpu

B = 16
H = 16
D = 64
HD = H * D
KV_SHARD = 1024
SPLIT = 4
KV_CHUNK = KV_SHARD // SPLIT
SCALE = D ** -0.5


def _head_expand_mask():
    col = lax.broadcasted_iota(jnp.int32, (H, HD), 1)
    row = lax.broadcasted_iota(jnp.int32, (H, HD), 0)
    return (col // D == row).astype(jnp.float32)


def _compute_body(p_ref, q_ref, k_ref, v_ref, o_ref, m_ref, l_ref):
    del p_ref
    q = q_ref[...]
    k2 = k_ref[0]
    v2 = v_ref[0]
    mask = _head_expand_mask()
    qbd = mask * q
    s = lax.dot_general(
        k2, qbd, (((1,), (1,)), ((), ())),
        preferred_element_type=jnp.float32,
    ) * SCALE
    m = jnp.max(s, axis=0, keepdims=True)
    p = jnp.exp(s - m)
    l = jnp.sum(p, axis=0, keepdims=True)
    o_full = lax.dot_general(
        p, v2, (((0,), (0,)), ((), ())),
        preferred_element_type=jnp.float32,
    )
    o_ref[...] = jnp.sum(o_full * mask, axis=0, keepdims=True)
    m_ref[...] = m
    l_ref[...] = l


def _combine_body(
    o_in, m_in, l_in, out_ref,
    acc_o, acc_ml, recv_o, recv_ml,
    send_o_sem, recv_o_sem, send_ml_sem, recv_ml_sem,
):
    my = [lax.axis_index(a) for a in ("x", "y", "z")]
    peers = []
    for ax in range(3):
        pc = list(my)
        pc[ax] = 1 - pc[ax]
        peers.append(tuple(pc))

    barrier = pltpu.get_barrier_semaphore()
    for pc in peers:
        pl.semaphore_signal(
            barrier, inc=1, device_id=pc,
            device_id_type=pl.DeviceIdType.MESH,
        )
    pl.semaphore_wait(barrier, 3)

    acc_o[...] = o_in[...]
    acc_ml[0] = m_in[...]
    acc_ml[1] = l_in[...]

    E = _head_expand_mask()

    def expand(a):
        return lax.dot_general(
            a, E, (((1,), (0,)), ((), ())),
            preferred_element_type=jnp.float32,
        )

    for step, pc in enumerate(peers):
        rdma_o = pltpu.make_async_remote_copy(
            src_ref=acc_o, dst_ref=recv_o.at[step],
            send_sem=send_o_sem.at[step], recv_sem=recv_o_sem.at[step],
            device_id=pc, device_id_type=pl.DeviceIdType.MESH,
        )
        rdma_ml = pltpu.make_async_remote_copy(
            src_ref=acc_ml, dst_ref=recv_ml.at[step],
            send_sem=send_ml_sem.at[step], recv_sem=recv_ml_sem.at[step],
            device_id=pc, device_id_type=pl.DeviceIdType.MESH,
        )
        rdma_o.start()
        rdma_ml.start()
        rdma_o.wait()
        rdma_ml.wait()

        m_a = acc_ml[0]
        l_a = acc_ml[1]
        m_b = recv_ml[step, 0]
        l_b = recv_ml[step, 1]
        m_n = jnp.maximum(m_a, m_b)
        ea = jnp.exp(m_a - m_n)
        eb = jnp.exp(m_b - m_n)
        acc_o[...] = acc_o[...] * expand(ea) + recv_o[step] * expand(eb)
        acc_ml[0] = m_n
        acc_ml[1] = l_a * ea + l_b * eb

    out_ref[...] = acc_o[...] / expand(acc_ml[1])


def kernel(Q, K, V):
    Q3 = Q.reshape(B, HD)
    K3 = K.reshape(B, KV_SHARD, HD)
    V3 = V.reshape(B, KV_SHARD, HD)

    p_idx = lax.axis_index("x") * 2 + lax.axis_index("z")
    p_arr = jnp.reshape(p_idx, (1,)).astype(jnp.int32)

    grid_spec = pltpu.PrefetchScalarGridSpec(
        num_scalar_prefetch=1,
        grid=(B,),
        in_specs=[
            pl.BlockSpec((1, HD), lambda b, p: (b, 0)),
            pl.BlockSpec((1, KV_CHUNK, HD), lambda b, p: (b, p[0], 0)),
            pl.BlockSpec((1, KV_CHUNK, HD), lambda b, p: (b, p[0], 0)),
        ],
        out_specs=[
            pl.BlockSpec((1, HD), lambda b, p: (b, 0)),
            pl.BlockSpec((1, H), lambda b, p: (b, 0)),
            pl.BlockSpec((1, H), lambda b, p: (b, 0)),
        ],
    )
    o_part, m_part, l_part = pl.pallas_call(
        _compute_body,
        grid_spec=grid_spec,
        out_shape=[
            jax.ShapeDtypeStruct((B, HD), jnp.float32),
            jax.ShapeDtypeStruct((B, H), jnp.float32),
            jax.ShapeDtypeStruct((B, H), jnp.float32),
        ],
    )(p_arr, Q3, K3, V3)

    out = pl.pallas_call(
        _combine_body,
        out_shape=jax.ShapeDtypeStruct((B, HD), jnp.float32),
        in_specs=[pl.BlockSpec(memory_space=pltpu.VMEM)] * 3,
        out_specs=pl.BlockSpec(memory_space=pltpu.VMEM),
        scratch_shapes=[
            pltpu.VMEM((B, HD), jnp.float32),
            pltpu.VMEM((2, B, H), jnp.float32),
            pltpu.VMEM((3, B, HD), jnp.float32),
            pltpu.VMEM((3, 2, B, H), jnp.float32),
            pltpu.SemaphoreType.DMA((3,)),
            pltpu.SemaphoreType.DMA((3,)),
            pltpu.SemaphoreType.DMA((3,)),
            pltpu.SemaphoreType.DMA((3,)),
        ],
        compiler_params=pltpu.CompilerParams(collective_id=0),
    )(o_part, m_part, l_part)

    return out.reshape(B, 1, H, D)


# baseline (device time: 163346 ns/iter reference)
import jax
import jax.numpy as jnp
from jax import lax
from jax.experimental import pallas as pl
from jax.experimental.pallas import tpu as pltpu

B = 16
H = 16
D = 64
HD = H * D
KV_SHARD = 1024
SPLIT = 4
KV_CHUNK = KV_SHARD // SPLIT
SCALE = D ** -0.5


def _head_expand_mask():
    col = lax.broadcasted_iota(jnp.int32, (H, HD), 1)
    row = lax.broadcasted_iota(jnp.int32, (H, HD), 0)
    return (col // D == row).astype(jnp.float32)


def _compute_body(p_ref, q_ref, k_ref, v_ref, o_ref, m_ref, l_ref):
    del p_ref
    q = q_ref[0]
    k2 = k_ref[0]
    v2 = v_ref[0]
    mask = _head_expand_mask()
    qbd = mask * q
    s = lax.dot_general(
        k2, qbd, (((1,), (1,)), ((), ())),
        preferred_element_type=jnp.float32,
    ) * SCALE
    m = jnp.max(s, axis=0, keepdims=True)
    p = jnp.exp(s - m)
    l = jnp.sum(p, axis=0, keepdims=True)
    o_full = lax.dot_general(
        p, v2, (((0,), (0,)), ((), ())),
        preferred_element_type=jnp.float32,
    )
    o_ref[0] = jnp.sum(o_full * mask, axis=0, keepdims=True)
    m_ref[0] = m
    l_ref[0] = l


def _combine_body(
    o_in, m_in, l_in, out_ref,
    acc_o, acc_ml, recv_o, recv_ml,
    send_o_sem, recv_o_sem, send_ml_sem, recv_ml_sem,
):
    my = [lax.axis_index(a) for a in ("x", "y", "z")]
    peers = []
    for ax in range(3):
        pc = list(my)
        pc[ax] = 1 - pc[ax]
        peers.append(tuple(pc))

    barrier = pltpu.get_barrier_semaphore()
    for pc in peers:
        pl.semaphore_signal(
            barrier, inc=1, device_id=pc,
            device_id_type=pl.DeviceIdType.MESH,
        )
    pl.semaphore_wait(barrier, 3)

    acc_o[...] = o_in[...]
    acc_ml[0] = m_in[...]
    acc_ml[1] = l_in[...]

    E = _head_expand_mask()

    def expand(a):
        return lax.dot_general(
            a, E, (((1,), (0,)), ((), ())),
            preferred_element_type=jnp.float32,
        )

    for step, pc in enumerate(peers):
        rdma_o = pltpu.make_async_remote_copy(
            src_ref=acc_o, dst_ref=recv_o.at[step],
            send_sem=send_o_sem.at[step], recv_sem=recv_o_sem.at[step],
            device_id=pc, device_id_type=pl.DeviceIdType.MESH,
        )
        rdma_ml = pltpu.make_async_remote_copy(
            src_ref=acc_ml, dst_ref=recv_ml.at[step],
            send_sem=send_ml_sem.at[step], recv_sem=recv_ml_sem.at[step],
            device_id=pc, device_id_type=pl.DeviceIdType.MESH,
        )
        rdma_o.start()
        rdma_ml.start()
        rdma_o.wait()
        rdma_ml.wait()

        m_a = acc_ml[0]
        l_a = acc_ml[1]
        m_b = recv_ml[step, 0]
        l_b = recv_ml[step, 1]
        m_n = jnp.maximum(m_a, m_b)
        ea = jnp.exp(m_a - m_n)
        eb = jnp.exp(m_b - m_n)
        acc_o[...] = acc_o[...] * expand(ea) + recv_o[step] * expand(eb)
        acc_ml[0] = m_n
        acc_ml[1] = l_a * ea + l_b * eb

    out_ref[...] = acc_o[...] / expand(acc_ml[1])


def kernel(Q, K, V):
    Q3 = Q.reshape(B, 1, HD)
    K3 = K.reshape(B, KV_SHARD, HD)
    V3 = V.reshape(B, KV_SHARD, HD)

    p_idx = lax.axis_index("x") * 2 + lax.axis_index("z")
    p_arr = jnp.reshape(p_idx, (1,)).astype(jnp.int32)

    grid_spec = pltpu.PrefetchScalarGridSpec(
        num_scalar_prefetch=1,
        grid=(B,),
        in_specs=[
            pl.BlockSpec((1, 1, HD), lambda b, p: (b, 0, 0)),
            pl.BlockSpec((1, KV_CHUNK, HD), lambda b, p: (b, p[0], 0)),
            pl.BlockSpec((1, KV_CHUNK, HD), lambda b, p: (b, p[0], 0)),
        ],
        out_specs=[
            pl.BlockSpec((1, 1, HD), lambda b, p: (b, 0, 0)),
            pl.BlockSpec((1, 1, H), lambda b, p: (b, 0, 0)),
            pl.BlockSpec((1, 1, H), lambda b, p: (b, 0, 0)),
        ],
    )
    o_part, m_part, l_part = pl.pallas_call(
        _compute_body,
        grid_spec=grid_spec,
        out_shape=[
            jax.ShapeDtypeStruct((B, 1, HD), jnp.float32),
            jax.ShapeDtypeStruct((B, 1, H), jnp.float32),
            jax.ShapeDtypeStruct((B, 1, H), jnp.float32),
        ],
    )(p_arr, Q3, K3, V3)
    o_part = o_part.reshape(B, HD)
    m_part = m_part.reshape(B, H)
    l_part = l_part.reshape(B, H)

    out = pl.pallas_call(
        _combine_body,
        out_shape=jax.ShapeDtypeStruct((B, HD), jnp.float32),
        in_specs=[pl.BlockSpec(memory_space=pltpu.VMEM)] * 3,
        out_specs=pl.BlockSpec(memory_space=pltpu.VMEM),
        scratch_shapes=[
            pltpu.VMEM((B, HD), jnp.float32),
            pltpu.VMEM((2, B, H), jnp.float32),
            pltpu.VMEM((3, B, HD), jnp.float32),
            pltpu.VMEM((3, 2, B, H), jnp.float32),
            pltpu.SemaphoreType.DMA((3,)),
            pltpu.SemaphoreType.DMA((3,)),
            pltpu.SemaphoreType.DMA((3,)),
            pltpu.SemaphoreType.DMA((3,)),
        ],
        compiler_params=pltpu.CompilerParams(collective_id=0),
    )(o_part, m_part, l_part)

    return out.reshape(B, 1, H, D)
